# baseline (speedup 1.0000x reference)
import functools

import jax
import jax.numpy as jnp
from jax import lax
from jax.experimental import pallas as pl
from jax.experimental.pallas import tpu as pltpu
from jax.experimental.pallas import tpu_sc as plsc

N_NODES = 10000
N_EDGES = 320000
D_FEAT = 128

NC = 2
NS = 16
L = 16
NW = NC * NS

CHUNK = 512
N_CHUNKS = N_EDGES // CHUNK
E_BIG = 20 * CHUNK
E_SMALL = 19 * CHUNK
N_BIG = N_CHUNKS - 19 * NW

TC_BLK = 10240


def _node_linear(x, wcat):
    def body(x_ref, w_ref, p_ref, c_ref):
        res = lax.dot_general(
            w_ref[...], x_ref[...],
            (((1,), (1,)), ((), ())),
            preferred_element_type=jnp.float32,
        )
        p_ref[...] = res[0]
        c_ref[...] = res[1]

    grid = (pl.cdiv(N_NODES, TC_BLK),)
    return pl.pallas_call(
        body,
        grid=grid,
        in_specs=[
            pl.BlockSpec((TC_BLK, D_FEAT), lambda i: (i, 0)),
            pl.BlockSpec((2, D_FEAT), lambda i: (0, 0)),
        ],
        out_specs=[
            pl.BlockSpec((TC_BLK,), lambda i: (i,)),
            pl.BlockSpec((TC_BLK,), lambda i: (i,)),
        ],
        out_shape=[
            jax.ShapeDtypeStruct((N_NODES,), jnp.float32),
            jax.ShapeDtypeStruct((N_NODES,), jnp.float32),
        ],
    )(x, wcat)


_sc_mesh = plsc.VectorSubcoreMesh(core_axis_name="c", subcore_axis_name="s")


@functools.partial(
    pl.kernel,
    mesh=_sc_mesh,
    out_type=jax.ShapeDtypeStruct((N_EDGES,), jnp.float32),
    compiler_params=pltpu.CompilerParams(
        needs_layout_passes=False, skip_device_barrier=True
    ),
    scratch_types=[
        pltpu.VMEM((N_NODES,), jnp.float32),
        pltpu.VMEM((N_NODES,), jnp.float32),
        pltpu.VMEM((2, E_BIG), jnp.int32),
        pltpu.VMEM((E_BIG,), jnp.float32),
        pltpu.VMEM((3 * L,), jnp.float32),
        pltpu.SemaphoreType.DMA,
    ],
)
def _edge_kernel(p_hbm, c_hbm, adjs_hbm, params_hbm, out_hbm,
                 p_v, c_v, sd_v, out_v, par_v, sem):
    wid = lax.axis_index("s") * NC + lax.axis_index("c")
    ebase = pl.multiple_of(
        (wid * 19 + jnp.minimum(wid, N_BIG)) * CHUNK, CHUNK
    )
    cps = [
        pltpu.async_copy(p_hbm, p_v, sem),
        pltpu.async_copy(c_hbm, c_v, sem),
        pltpu.async_copy(params_hbm, par_v, sem),
    ]

    def run(nedges):
        pltpu.sync_copy(
            adjs_hbm.at[:, pl.ds(ebase, nedges)],
            sd_v.at[:, pl.ds(0, nedges)],
        )
        wv = par_v[pl.ds(0, L)]
        bv = par_v[pl.ds(L, L)]
        dv = par_v[pl.ds(2 * L, L)]

        @plsc.parallel_loop(0, nedges, step=L, unroll=16)
        def _(e0):
            sl = pl.ds(e0, L)
            gp = plsc.load_gather(p_v, [sd_v[1, sl]])
            gc = plsc.load_gather(c_v, [sd_v[0, sl]])
            out_v[sl] = jnp.abs(gp - gc + dv) * wv + bv

        pltpu.sync_copy(
            out_v.at[pl.ds(0, nedges)], out_hbm.at[pl.ds(ebase, nedges)]
        )

    for cp in cps:
        cp.wait()

    @pl.when(wid < N_BIG)
    def _():
        run(E_BIG)

    @pl.when(wid >= N_BIG)
    def _():
        run(E_SMALL)


def kernel(x, adjs, Wp, bp, Wc, bc, W1, b1):
    wcat = jnp.concatenate([Wp, Wc], axis=1).T
    p, c = _node_linear(x, wcat)
    adjs32 = adjs.astype(jnp.int32)
    ones = jnp.ones((L,), jnp.float32)
    params = jnp.concatenate([
        W1.reshape(()) * ones,
        b1.reshape(()) * ones,
        (bp - bc).reshape(()) * ones,
    ])
    return _edge_kernel(p, c, adjs32, params)


# speedup vs baseline: 1.0223x; 1.0223x over previous
import functools

import jax
import jax.numpy as jnp
from jax import lax
from jax.experimental import pallas as pl
from jax.experimental.pallas import tpu as pltpu
from jax.experimental.pallas import tpu_sc as plsc

N_NODES = 10000
N_EDGES = 320000
D_FEAT = 128

NC = 2
NS = 16
L = 16
NW = NC * NS

CHUNK = 512
N_CHUNKS = N_EDGES // CHUNK
E_BIG = 20 * CHUNK
E_SMALL = 19 * CHUNK
N_BIG = N_CHUNKS - 19 * NW

TC_BLK = 10240


def _node_linear(x, wcat):
    def body(x_ref, w_ref, p_ref, c_ref):
        res = lax.dot_general(
            w_ref[...], x_ref[...],
            (((1,), (1,)), ((), ())),
            preferred_element_type=jnp.float32,
        )
        p_ref[...] = res[0]
        c_ref[...] = res[1]

    grid = (pl.cdiv(N_NODES, TC_BLK),)
    return pl.pallas_call(
        body,
        grid=grid,
        in_specs=[
            pl.BlockSpec((TC_BLK, D_FEAT), lambda i: (i, 0)),
            pl.BlockSpec((2, D_FEAT), lambda i: (0, 0)),
        ],
        out_specs=[
            pl.BlockSpec((TC_BLK,), lambda i: (i,)),
            pl.BlockSpec((TC_BLK,), lambda i: (i,)),
        ],
        out_shape=[
            jax.ShapeDtypeStruct((N_NODES,), jnp.float32),
            jax.ShapeDtypeStruct((N_NODES,), jnp.float32),
        ],
    )(x, wcat)


_sc_mesh = plsc.VectorSubcoreMesh(core_axis_name="c", subcore_axis_name="s")


@functools.partial(
    pl.kernel,
    mesh=_sc_mesh,
    out_type=jax.ShapeDtypeStruct((N_EDGES,), jnp.float32),
    compiler_params=pltpu.CompilerParams(needs_layout_passes=False),
    scratch_types=[
        pltpu.VMEM((N_NODES,), jnp.float32),
        pltpu.VMEM((N_NODES,), jnp.float32),
        pltpu.VMEM((2, E_BIG), jnp.int32),
        pltpu.VMEM((E_BIG,), jnp.float32),
        pltpu.VMEM((3 * L,), jnp.float32),
        pltpu.SemaphoreType.DMA,
    ],
)
def _edge_kernel(p_hbm, c_hbm, adjs_hbm, params_hbm, out_hbm,
                 p_v, c_v, sd_v, out_v, par_v, sem):
    wid = lax.axis_index("s") * NC + lax.axis_index("c")
    ebase = pl.multiple_of(
        (wid * 19 + jnp.minimum(wid, N_BIG)) * CHUNK, CHUNK
    )
    cps = [
        pltpu.async_copy(p_hbm, p_v, sem),
        pltpu.async_copy(c_hbm, c_v, sem),
        pltpu.async_copy(params_hbm, par_v, sem),
    ]

    def run(nedges):
        wv = par_v[pl.ds(0, L)]
        bv = par_v[pl.ds(L, L)]
        dv = par_v[pl.ds(2 * L, L)]

        @plsc.parallel_loop(0, nedges, step=L, unroll=16)
        def _(e0):
            sl = pl.ds(e0, L)
            gp = plsc.load_gather(p_v, [sd_v[1, sl]])
            gc = plsc.load_gather(c_v, [sd_v[0, sl]])
            out_v[sl] = jnp.abs(gp - gc + dv) * wv + bv

        pltpu.sync_copy(
            out_v.at[pl.ds(0, nedges)], out_hbm.at[pl.ds(ebase, nedges)]
        )

    @pl.when(wid < N_BIG)
    def _():
        cp = pltpu.async_copy(
            adjs_hbm.at[:, pl.ds(ebase, E_BIG)],
            sd_v.at[:, pl.ds(0, E_BIG)], sem,
        )
        for c0 in cps:
            c0.wait()
        cp.wait()
        run(E_BIG)

    @pl.when(wid >= N_BIG)
    def _():
        cp = pltpu.async_copy(
            adjs_hbm.at[:, pl.ds(ebase, E_SMALL)],
            sd_v.at[:, pl.ds(0, E_SMALL)], sem,
        )
        for c0 in cps:
            c0.wait()
        cp.wait()
        run(E_SMALL)


def kernel(x, adjs, Wp, bp, Wc, bc, W1, b1):
    wcat = jnp.concatenate([Wp, Wc], axis=1).T
    p, c = _node_linear(x, wcat)
    adjs32 = adjs.astype(jnp.int32)
    ones = jnp.ones((L,), jnp.float32)
    params = jnp.concatenate([
        W1.reshape(()) * ones,
        b1.reshape(()) * ones,
        (bp - bc).reshape(()) * ones,
    ])
    return _edge_kernel(p, c, adjs32, params)
